# Initial kernel scaffold; baseline (speedup 1.0000x reference)
import jax
import jax.numpy as jnp
from jax.experimental import pallas as pl


def kernel(logits, features, W1, b1, W2, b2):
    raise NotImplementedError("write your pallas kernel here")


# speedup vs baseline: 2.2473x; 2.2473x over previous
import jax
import jax.numpy as jnp
from jax import lax
from jax.experimental import pallas as pl
from jax.experimental.pallas import tpu as pltpu
from jax.experimental.pallas import tpu_sc as plsc

NC, NS, L = 2, 16, 16
NW = NC * NS
ROWS, COLS = 128, 32768
RPW = ROWS // NW
NVEC = COLS // L


def _insert5(ms, v):
    out = []
    for i in range(4):
        hi = jnp.maximum(ms[i], v)
        v = jnp.minimum(ms[i], v)
        out.append(hi)
    out.append(jnp.maximum(ms[4], v))
    return out


def _merge5(a, b):
    r = []
    for k in range(5):
        cur = jnp.maximum(a[k], b[k])
        for i in range(k):
            cur = jnp.maximum(cur, jnp.minimum(a[i], b[k - 1 - i]))
        r.append(cur)
    return r


def _body(logits_ref, wtab_ref, out_ref,
          buf0, buf1, wbuf, cand, obuf, sem0, sem1, wsem):
    wid = lax.axis_index("s") * NC + lax.axis_index("c")
    base = wid * RPW
    bufs = [buf0, buf1]
    sems = [sem0, sem1]

    wcopy = pltpu.async_copy(wtab_ref, wbuf, wsem)
    descs = [
        pltpu.async_copy(logits_ref.at[base + 0], buf0, sem0),
        pltpu.async_copy(logits_ref.at[base + 1], buf1, sem1),
    ]
    wcopy.wait()

    neg = jnp.full((L,), -jnp.inf, dtype=jnp.float32)
    lane = lax.iota(jnp.int32, L)
    T = [neg] * 5

    for j in range(RPW):
        b = bufs[j % 2]
        descs[j % 2].wait()

        def step(k, ms, b=b):
            v = b[pl.ds(k * L, L)]
            return tuple(_insert5(list(ms), v))

        ms = list(lax.fori_loop(0, NVEC, step, (neg,) * 5, unroll=8))

        if j + 2 < RPW:
            descs[j % 2] = pltpu.async_copy(
                logits_ref.at[base + j + 2], bufs[j % 2], sems[j % 2])

        for s in (1, 2, 4, 8):
            perm = jnp.bitwise_xor(lane, s)
            part = []
            for i in range(5):
                cand[...] = ms[i]
                part.append(plsc.load_gather(cand, [perm]))
            ms = _merge5(ms, part)

        is_j = lane == j
        for i in range(5):
            T[i] = jnp.where(is_j, ms[i], T[i])

    h = []
    for jo in range(5):
        acc = wbuf[25 + jo]
        for i in range(5):
            acc = acc + wbuf[jo * 5 + i] * T[i]
        h.append(jnp.maximum(acc, 0.0))
    z = wbuf[35]
    for jo in range(5):
        z = z + wbuf[30 + jo] * h[jo]
    obuf[...] = 1.0 / (1.0 + jnp.exp(-z))
    pltpu.sync_copy(obuf, out_ref.at[wid])


@jax.jit
def _run(logits, wtab):
    mesh = plsc.VectorSubcoreMesh(
        core_axis_name="c", subcore_axis_name="s",
        num_cores=NC, num_subcores=NS)
    f = pl.kernel(
        _body,
        out_type=jax.ShapeDtypeStruct((NW, L), jnp.float32),
        mesh=mesh,
        compiler_params=pltpu.CompilerParams(needs_layout_passes=False),
        scratch_types=[
            pltpu.VMEM((COLS,), jnp.float32),
            pltpu.VMEM((COLS,), jnp.float32),
            pltpu.VMEM((36, L), jnp.float32),
            pltpu.VMEM((L,), jnp.float32),
            pltpu.VMEM((L,), jnp.float32),
            pltpu.SemaphoreType.DMA,
            pltpu.SemaphoreType.DMA,
            pltpu.SemaphoreType.DMA,
        ],
    )
    return f(logits, wtab)


def kernel(logits, features, W1, b1, W2, b2):
    del features
    wvec = jnp.concatenate([
        W1.reshape(-1), b1.reshape(-1), W2.reshape(-1), b2.reshape(-1)])
    wtab = jnp.broadcast_to(wvec[:, None], (36, L)).astype(jnp.float32)
    out32 = _run(logits, wtab)
    return out32[:, :RPW].reshape(ROWS, 1)
